# Initial kernel scaffold; baseline (speedup 1.0000x reference)
import jax
import jax.numpy as jnp
from jax.experimental import pallas as pl


def kernel(x, neigh_orders):
    raise NotImplementedError("write your pallas kernel here")


# speedup vs baseline: 14.6780x; 14.6780x over previous
import functools

import jax
import jax.numpy as jnp
from jax import lax
from jax.experimental import pallas as pl
from jax.experimental.pallas import tpu as pltpu
from jax.experimental.pallas import tpu_sc as plsc

N_IN = 163842
D = 512
NUM_NODES = (N_IN + 6) // 4

NB = 8
NBLK = (NUM_NODES + NB - 1) // NB
NPAD = NBLK * NB
NC = 2
NS = 16
NW = NC * NS


@functools.partial(
    pl.kernel,
    mesh=plsc.VectorSubcoreMesh(core_axis_name="c", subcore_axis_name="s"),
    compiler_params=pltpu.CompilerParams(needs_layout_passes=False),
    out_type=jax.ShapeDtypeStruct((NPAD * D,), jnp.float32),
    scratch_types=[
        pltpu.VMEM((NB * 7,), jnp.int32),
        pltpu.VMEM((NB * 7, D), jnp.float32),
        pltpu.VMEM((NB * D,), jnp.float32),
        pltpu.SemaphoreType.DMA,
    ],
)
def _pool_sc(x_hbm, idx_hbm, out_hbm, idx_v, rows_v, out_v, sem):
    wid = lax.axis_index("s") * NC + lax.axis_index("c")
    nblk_w = (NBLK - wid + NW - 1) // NW
    iota7 = jax.lax.iota(jnp.int32, 16) * 7

    def chunk_body(t, _):
        n = t >> 5
        j = t & 31
        p0 = n * (7 * D) + j * 112
        pv = jnp.full((16,), p0, jnp.int32) + iota7
        acc = jnp.zeros((16,), jnp.float32)
        for k in range(7):
            pk = pv + k
            acc = acc + plsc.load_gather(rows_v, [pk >> 9, pk & 511])
        out_v[pl.ds(n * D + j * 16, 16)] = acc * (1.0 / 7.0)
        return 0

    def blk_body(i, _):
        b = wid + i * NW
        node_base = b * NB
        pltpu.sync_copy(idx_hbm.at[pl.ds(node_base * 7, NB * 7)], idx_v)
        pltpu.async_copy(x_hbm.at[idx_v], rows_v, sem).wait()
        lax.fori_loop(0, NB * 32, chunk_body, 0)
        pltpu.sync_copy(out_v, out_hbm.at[pl.ds(node_base * D, NB * D)])
        return 0

    lax.fori_loop(0, nblk_w, blk_body, 0)


def kernel(x, neigh_orders):
    idx = neigh_orders[: NUM_NODES * 7]
    pad = NPAD * 7 - NUM_NODES * 7
    idx = jnp.concatenate([idx, jnp.zeros((pad,), jnp.int32)])
    out_flat = _pool_sc(x, idx)
    return out_flat.reshape(NPAD, D)[:NUM_NODES]


# speedup vs baseline: 18.3803x; 1.2522x over previous
import functools

import jax
import jax.numpy as jnp
from jax import lax
from jax.experimental import pallas as pl
from jax.experimental.pallas import tpu as pltpu
from jax.experimental.pallas import tpu_sc as plsc

N_IN = 163842
D = 512
NUM_NODES = (N_IN + 6) // 4

NB = 8
NC = 2
NS = 16
NW = NC * NS
NBW = 162
NBLK = NW * NBW
NPAD = NBLK * NB
IDX_W = NBW * NB * 7
IDX_FETCH = IDX_W + 2 * NB * 7
IDX_PAD = (NW - 1) * IDX_W + IDX_FETCH


@functools.partial(
    pl.kernel,
    mesh=plsc.VectorSubcoreMesh(core_axis_name="c", subcore_axis_name="s"),
    compiler_params=pltpu.CompilerParams(needs_layout_passes=False),
    out_type=jax.ShapeDtypeStruct((NPAD * D,), jnp.float32),
    scratch_types=[
        pltpu.VMEM((IDX_FETCH,), jnp.int32),
        pltpu.VMEM((NB * 7, D), jnp.float32),
        pltpu.VMEM((NB * 7, D), jnp.float32),
        pltpu.VMEM((NB * D,), jnp.float32),
        pltpu.VMEM((NB * D,), jnp.float32),
        pltpu.SemaphoreType.DMA,
        pltpu.SemaphoreType.DMA,
        pltpu.SemaphoreType.DMA,
        pltpu.SemaphoreType.DMA,
    ],
)
def _pool_sc(x_hbm, idx_hbm, out_hbm, idx_v, rows0, rows1, o0, o1,
             rsem0, rsem1, osem0, osem1):
    wid = lax.axis_index("s") * NC + lax.axis_index("c")
    iota7 = lax.iota(jnp.int32, 16) * 7
    bases = [iota7 + n * (7 * D) for n in range(NB)]

    pltpu.sync_copy(idx_hbm.at[pl.ds(wid * IDX_W, IDX_FETCH)], idx_v)

    def idx_slice(blk):
        return idx_v.at[pl.ds(blk * (NB * 7), NB * 7)]

    def gather_rows(blk, buf, sem):
        return pltpu.async_copy(x_hbm.at[idx_slice(blk)], buf, sem)

    def wait_rows(blk, buf, sem):
        pltpu.make_async_copy(x_hbm.at[idx_slice(blk)], buf, sem).wait()

    def pool_block(buf, out_v):
        def chunk(j, _):
            off = j * 112
            for n in range(NB):
                pv = bases[n] + off
                acc = plsc.load_gather(buf, [pv >> 9, pv & 511])
                for k in range(1, 7):
                    pk = pv + k
                    acc = acc + plsc.load_gather(buf, [pk >> 9, pk & 511])
                out_v[pl.ds(n * D + j * 16, 16)] = acc * (1.0 / 7.0)
            return 0

        lax.fori_loop(0, 32, chunk, 0)

    def out_ref(blk):
        g = wid * NBW + blk
        return out_hbm.at[pl.ds(g * (NB * D), NB * D)]

    def copy_out(blk, out_v, sem):
        return pltpu.async_copy(out_v, out_ref(blk), sem)

    def wait_out(blk, out_v, sem):
        pltpu.make_async_copy(out_v, out_ref(blk), sem).wait()

    gather_rows(0, rows0, rsem0)
    gather_rows(1, rows1, rsem1)

    def step(t, _):
        b0 = 2 * t
        wait_rows(b0, rows0, rsem0)
        pool_block(rows0, o0)
        gather_rows(b0 + 2, rows0, rsem0)
        copy_out(b0, o0, osem0)

        wait_rows(b0 + 1, rows1, rsem1)
        pool_block(rows1, o1)
        gather_rows(b0 + 3, rows1, rsem1)
        copy_out(b0 + 1, o1, osem1)

        wait_out(b0, o0, osem0)
        wait_out(b0 + 1, o1, osem1)
        return 0

    lax.fori_loop(0, NBW // 2, step, 0)


def kernel(x, neigh_orders):
    idx = neigh_orders[: NUM_NODES * 7]
    idx = jnp.concatenate(
        [idx, jnp.zeros((IDX_PAD - NUM_NODES * 7,), jnp.int32)])
    out_flat = _pool_sc(x, idx)
    return out_flat.reshape(NPAD, D)[:NUM_NODES]
